# Initial kernel scaffold; baseline (speedup 1.0000x reference)
import jax
import jax.numpy as jnp
from jax.experimental import pallas as pl


def kernel(x, W1, b1, W2, b2, emb):
    raise NotImplementedError("write your pallas kernel here")


# speedup vs baseline: 3.0981x; 3.0981x over previous
import jax
import jax.numpy as jnp
from jax.experimental import pallas as pl

BIN = 100
PAD = 128
HID = 128
TB = 2048


def _body(x_ref, w1_ref, b1_ref, w2_ref, b2_ref, emb_ref, out_ref):
    xb = x_ref[...]
    h1 = jnp.maximum(xb * w1_ref[...] + b1_ref[...], 0.0)
    h2 = jax.lax.dot_general(
        h1, w2_ref[...], (((1,), (0,)), ((), ())),
        precision=jax.lax.Precision.DEFAULT,
        preferred_element_type=jnp.float32,
    ) + b2_ref[...]
    h2 = jnp.maximum(h2, 0.0)
    m = jnp.max(h2, axis=1, keepdims=True)
    lane = jax.lax.broadcasted_iota(jnp.int32, (TB, PAD), 1)
    idx = jnp.min(jnp.where(h2 >= m, lane, PAD), axis=1, keepdims=True)
    onehot = (lane == idx).astype(jnp.float32)
    out_ref[...] = jax.lax.dot_general(
        onehot, emb_ref[...], (((1,), (0,)), ((), ())),
        precision=jax.lax.Precision.HIGHEST,
        preferred_element_type=jnp.float32,
    )


def kernel(x, W1, b1, W2, b2, emb):
    B, L, _ = x.shape
    N = B * L
    xf = x.reshape(N, 1)
    w1p = jnp.zeros((1, PAD), jnp.float32).at[:, :BIN].set(W1)
    b1p = jnp.zeros((1, PAD), jnp.float32).at[:, :BIN].set(b1)
    w2p = jnp.zeros((PAD, PAD), jnp.float32).at[:BIN, :BIN].set(W2)
    b2p = jnp.zeros((1, PAD), jnp.float32).at[:, :BIN].set(b2)
    embp = jnp.zeros((PAD, HID), jnp.float32).at[:BIN, :].set(emb)

    grid = N // TB
    out = pl.pallas_call(
        _body,
        grid=(grid,),
        in_specs=[
            pl.BlockSpec((TB, 1), lambda i: (i, 0)),
            pl.BlockSpec((1, PAD), lambda i: (0, 0)),
            pl.BlockSpec((1, PAD), lambda i: (0, 0)),
            pl.BlockSpec((PAD, PAD), lambda i: (0, 0)),
            pl.BlockSpec((1, PAD), lambda i: (0, 0)),
            pl.BlockSpec((PAD, HID), lambda i: (0, 0)),
        ],
        out_specs=pl.BlockSpec((TB, HID), lambda i: (i, 0)),
        out_shape=jax.ShapeDtypeStruct((N, HID), jnp.float32),
    )(xf, w1p, b1p, w2p, b2p, embp)
    return out.reshape(B, L, HID)
